# baseline (speedup 1.0000x reference)
import functools

import jax
import jax.numpy as jnp
from jax import lax
from jax.experimental import pallas as pl
from jax.experimental.pallas import tpu as pltpu
from jax.experimental.pallas import tpu_sc as plsc

TOPK_K = 128
N_ROWS = 24576
ROW_LEN = 576
LANES = 16
NVEC = ROW_LEN // LANES
N_WORKERS = 32
ROWS_PER_W = N_ROWS // N_WORKERS
CHUNK = 128
N_CHUNKS = ROWS_PER_W // CHUNK
INT_MIN32 = -(2**31)


def _srl(x, n):
    return lax.shift_right_logical(x, n)


def _sc_body(x_hbm, o_hbm, buf, hist):
    c = lax.axis_index("c")
    s = lax.axis_index("s")
    wid = s * 2 + c
    row0 = wid * ROWS_PER_W

    def chunk_body(ci, carry):
        base = row0 + ci * CHUNK
        pltpu.sync_copy(x_hbm.at[pl.ds(base, CHUNK)], buf)

        def row_body(r, rcarry):
            keys = []
            for j in range(NVEC):
                b = buf[r, pl.ds(j * LANES, LANES)]
                ubkey = b ^ ((b >> 31) | INT_MIN32)
                keys.append(ubkey)

            zeros = jnp.zeros((LANES,), jnp.int32)

            four = jnp.full((LANES,), 4, jnp.int32)

            def pass_body(p, pk):
                prefix, krem = pk
                shift = jnp.full((LANES,), 28 - 4 * p, jnp.int32)
                hist[pl.ds(0, LANES)] = zeros
                for kj in keys:
                    hi = _srl(kj, shift)
                    d = hi & jnp.int32(0xF)
                    m = _srl(hi, four) == prefix
                    cnts, last = plsc.scan_count(d, mask=m)
                    plsc.addupdate_scatter(hist, [d], cnts, mask=last)
                h = hist[pl.ds(0, LANES)]
                cum = plsc.cumsum(lax.rev(h, (0,)))
                l = plsc.all_reduce_population_count(cum < krem)
                digit = 15 - l
                lm1 = jnp.maximum(l - 1, 0)
                above = jnp.where(l == 0, zeros, jnp.take(cum, lm1))
                return (prefix << four) | digit, krem - above

            prefix, _ = lax.fori_loop(
                0, 8, pass_body, (zeros, jnp.full((LANES,), TOPK_K, jnp.int32))
            )
            t_signed = prefix ^ INT_MIN32
            for j in range(NVEC):
                bv = buf[r, pl.ds(j * LANES, LANES)]
                skey = keys[j] ^ INT_MIN32
                buf[r, pl.ds(j * LANES, LANES)] = jnp.where(
                    skey >= t_signed, bv, zeros
                )
            return rcarry

        lax.fori_loop(0, CHUNK, row_body, 0)
        pltpu.sync_copy(buf, o_hbm.at[pl.ds(base, CHUNK)])
        return carry

    lax.fori_loop(0, N_CHUNKS, chunk_body, 0)


@jax.jit
def _sc_sparsify(xr):
    mesh = plsc.VectorSubcoreMesh(core_axis_name="c", subcore_axis_name="s")
    fn = pl.kernel(
        _sc_body,
        out_type=jax.ShapeDtypeStruct((N_ROWS, ROW_LEN), jnp.int32),
        mesh=mesh,
        compiler_params=pltpu.CompilerParams(needs_layout_passes=False),
        scratch_types=[
            pltpu.VMEM((CHUNK, ROW_LEN), jnp.int32),
            pltpu.VMEM((LANES,), jnp.int32),
        ],
    )
    return fn(xr)


def kernel(x):
    n, c, h, w = x.shape
    xr = lax.bitcast_convert_type(x.reshape(n * c, h * w), jnp.int32)
    out = _sc_sparsify(xr)
    return lax.bitcast_convert_type(out, jnp.float32).reshape(n, c, h, w)
